# baseline (speedup 1.0000x reference)
import jax
import jax.numpy as jnp
from jax import lax
from jax.experimental import pallas as pl
from jax.experimental.pallas import tpu as pltpu

_NQ = 12
_NL = 6
_NC = 64
_B = 512
_BBLK = 256
_NA = 1 << _NQ


def _qnn_body(xt_ref, qw_ref, w_ref, b_ref, out_ref):
    f32 = jnp.float32

    xh = xt_ref[...] * 0.5
    cx = jnp.cos(xh)
    sx = jnp.sin(xh)
    qh = qw_ref[...] * 0.5
    cq = jnp.cos(qh)
    sq = jnp.sin(qh)

    miota = lax.broadcasted_iota(jnp.int32, (1, 8, _BBLK), 1)
    mb0 = (miota & 1) == 1
    mb1 = (miota & 2) == 2
    mb2 = (miota & 4) == 4
    mb = (mb0, mb1, mb2)

    def partner(a, bit):
        if bit >= 3:
            s = 1 << bit
            y = a.reshape(_NA // (2 * s), 2, s, _BBLK)
            p = jnp.concatenate([y[:, 1:2], y[:, 0:1]], axis=1)
            return p.reshape(_NA, _BBLK)
        y = a.reshape(_NA // 8, 8, _BBLK)
        if bit == 2:
            p = jnp.roll(y, 4, axis=1)
        else:
            s = 1 << bit
            p = jnp.where(mb[bit], jnp.roll(y, s, axis=1),
                          jnp.roll(y, -s, axis=1))
        return p.reshape(_NA, _BBLK)

    def rx(r, im, c, s, bit):
        pr = partner(r, bit)
        pi = partner(im, bit)
        return c * r + s * pi, c * im - s * pr

    def cnot_comp(a, w):
        if w <= 7:
            cb = 11 - w
            t = 1 << (cb - 1)
            y = a.reshape(_NA >> (cb + 1), 4, t, _BBLK)
            p = jnp.concatenate([y[:, 0:2], y[:, 3:4], y[:, 2:3]], axis=1)
            return p.reshape(_NA, _BBLK)
        if w == 8:
            y = a.reshape(_NA // 16, 2, 8, _BBLK)
            y1 = jnp.roll(y[:, 1], 4, axis=1).reshape(_NA // 16, 1, 8, _BBLK)
            return jnp.concatenate([y[:, 0:1], y1], axis=1).reshape(_NA, _BBLK)
        y = a.reshape(_NA // 8, 8, _BBLK)
        if w == 9:
            p = jnp.where(mb1, jnp.roll(y, 2, axis=1), jnp.roll(y, -2, axis=1))
            return jnp.where(mb2, p, y).reshape(_NA, _BBLK)
        if w == 10:
            p = jnp.where(mb0, jnp.roll(y, 1, axis=1), jnp.roll(y, -1, axis=1))
            return jnp.where(mb1, p, y).reshape(_NA, _BBLK)
        h = a.reshape(2, _NA // 16, 8, _BBLK)
        p = jnp.concatenate([h[1:2], h[0:1]], axis=0).reshape(_NA // 8, 8, _BBLK)
        return jnp.where(mb0, p, y).reshape(_NA, _BBLK)

    r = jnp.ones((1, _BBLK), f32)
    im = jnp.zeros((1, _BBLK), f32)
    for k in range(_NQ):
        w = _NQ - 1 - k
        c = cx[w:w + 1, :]
        s = sx[w:w + 1, :]
        nr = jnp.concatenate([c * r, s * im], axis=0)
        ni = jnp.concatenate([c * im, -(s * r)], axis=0)
        r, im = nr, ni

    jx = lax.broadcasted_iota(jnp.int32, (64, 64), 0)
    kx = lax.broadcasted_iota(jnp.int32, (64, 64), 1)
    xk = jx ^ kx
    xbits = [(xk >> (5 - w)) & 1 == 1 for w in range(6)]
    pc = lax.population_count(xk)
    podd = (pc & 1) == 1
    ptwo = (pc & 2) == 2
    dn = (((1,), (0,)), ((), ()))

    for l in range(_NL):
        for w in range(6, _NQ):
            r, im = rx(r, im, cq[l, w], sq[l, w], _NQ - 1 - w)
        amag = jnp.float32(1.0)
        for w in range(6):
            amag = amag * jnp.where(xbits[w], sq[l, w], cq[l, w])
        ar = jnp.where(podd, 0.0, jnp.where(ptwo, -amag, amag))
        ai = jnp.where(podd, jnp.where(ptwo, amag, -amag), 0.0)
        r3 = r.reshape(64, 64, _BBLK)
        i3 = im.reshape(64, 64, _BBLK)
        hp = lax.Precision.HIGHEST
        rr = lax.dot_general(ar, r3, dn, precision=hp, preferred_element_type=f32)
        ri = lax.dot_general(ai, i3, dn, precision=hp, preferred_element_type=f32)
        ir = lax.dot_general(ar, i3, dn, precision=hp, preferred_element_type=f32)
        ii = lax.dot_general(ai, r3, dn, precision=hp, preferred_element_type=f32)
        r = (rr - ri).reshape(_NA, _BBLK)
        im = (ir + ii).reshape(_NA, _BBLK)
        for w in range(_NQ):
            r = cnot_comp(r, w)
            im = cnot_comp(im, w)

    probs = r * r + im * im

    col = lax.broadcasted_iota(jnp.int32, (_NQ, _NA), 1)
    row = lax.broadcasted_iota(jnp.int32, (_NQ, _NA), 0)
    bitpos = jnp.right_shift(jnp.int32(_NA // 2), row)
    sgn = jnp.where((col & bitpos) != 0, -1.0, 1.0).astype(f32)
    z = jnp.dot(sgn, probs, preferred_element_type=f32)
    logits = jnp.dot(w_ref[...], z, preferred_element_type=f32)
    out_ref[...] = logits + b_ref[...]


def kernel(x, qweights, W, b):
    xt = x.T
    b2 = b.reshape(_NC, 1)
    out = pl.pallas_call(
        _qnn_body,
        grid=(_B // _BBLK,),
        in_specs=[
            pl.BlockSpec((_NQ, _BBLK), lambda i: (0, i)),
            pl.BlockSpec((_NL, _NQ), lambda i: (0, 0)),
            pl.BlockSpec((_NC, _NQ), lambda i: (0, 0)),
            pl.BlockSpec((_NC, 1), lambda i: (0, 0)),
        ],
        out_specs=pl.BlockSpec((_NC, _BBLK), lambda i: (0, i)),
        out_shape=jax.ShapeDtypeStruct((_NC, _B), jnp.float32),
        compiler_params=pltpu.CompilerParams(
            dimension_semantics=("parallel",),
            vmem_limit_bytes=100 * 1024 * 1024,
        ),
    )(xt, qweights, W, b2)
    return out.T


# speedup vs baseline: 1.2679x; 1.2679x over previous
import jax
import jax.numpy as jnp
from jax import lax
from jax.experimental import pallas as pl
from jax.experimental.pallas import tpu as pltpu

_NQ = 12
_NL = 6
_NC = 64
_B = 512
_BBLK = 256
_NA = 1 << _NQ


def _qnn_body(xt_ref, qw_ref, w_ref, b_ref, out_ref):
    f32 = jnp.float32

    xh = xt_ref[...] * 0.5
    cx = jnp.cos(xh)
    sx = jnp.sin(xh)
    qh = qw_ref[...] * 0.5
    cq = jnp.cos(qh)
    sq = jnp.sin(qh)

    miota = lax.broadcasted_iota(jnp.int32, (1, 8, _BBLK), 1)
    mb0 = (miota & 1) == 1
    mb1 = (miota & 2) == 2
    mb2 = (miota & 4) == 4
    mb = (mb0, mb1, mb2)

    def partner(a, bit):
        if bit >= 3:
            s = 1 << bit
            y = a.reshape(_NA // (2 * s), 2, s, _BBLK)
            p = jnp.concatenate([y[:, 1:2], y[:, 0:1]], axis=1)
            return p.reshape(_NA, _BBLK)
        y = a.reshape(_NA // 8, 8, _BBLK)
        if bit == 2:
            p = jnp.roll(y, 4, axis=1)
        else:
            s = 1 << bit
            p = jnp.where(mb[bit], jnp.roll(y, s, axis=1),
                          jnp.roll(y, -s, axis=1))
        return p.reshape(_NA, _BBLK)

    def rx(r, im, c, s, bit):
        pr = partner(r, bit)
        pi = partner(im, bit)
        return c * r + s * pi, c * im - s * pr

    def cnot_comp(a, w):
        if w <= 7:
            cb = 11 - w
            t = 1 << (cb - 1)
            y = a.reshape(_NA >> (cb + 1), 4, t, _BBLK)
            p = jnp.concatenate([y[:, 0:2], y[:, 3:4], y[:, 2:3]], axis=1)
            return p.reshape(_NA, _BBLK)
        if w == 8:
            y = a.reshape(_NA // 16, 2, 8, _BBLK)
            y1 = jnp.roll(y[:, 1], 4, axis=1).reshape(_NA // 16, 1, 8, _BBLK)
            return jnp.concatenate([y[:, 0:1], y1], axis=1).reshape(_NA, _BBLK)
        y = a.reshape(_NA // 8, 8, _BBLK)
        if w == 9:
            p = jnp.where(mb1, jnp.roll(y, 2, axis=1), jnp.roll(y, -2, axis=1))
            return jnp.where(mb2, p, y).reshape(_NA, _BBLK)
        if w == 10:
            p = jnp.where(mb0, jnp.roll(y, 1, axis=1), jnp.roll(y, -1, axis=1))
            return jnp.where(mb1, p, y).reshape(_NA, _BBLK)
        h = a.reshape(2, _NA // 16, 8, _BBLK)
        p = jnp.concatenate([h[1:2], h[0:1]], axis=0).reshape(_NA // 8, 8, _BBLK)
        return jnp.where(mb0, p, y).reshape(_NA, _BBLK)

    r = jnp.ones((1, _BBLK), f32)
    im = jnp.zeros((1, _BBLK), f32)
    for k in range(_NQ):
        w = _NQ - 1 - k
        c = cx[w:w + 1, :]
        s = sx[w:w + 1, :]
        nr = jnp.concatenate([c * r, s * im], axis=0)
        ni = jnp.concatenate([c * im, -(s * r)], axis=0)
        r, im = nr, ni

    jx = lax.broadcasted_iota(jnp.int32, (64, 64), 0)
    kx = lax.broadcasted_iota(jnp.int32, (64, 64), 1)
    xk = jx ^ kx
    xbits = [(xk >> (5 - w)) & 1 == 1 for w in range(6)]
    sgnb_par = (lax.population_count(kx) + lax.population_count(jx & kx)) & 1
    sgnb = jnp.where(sgnb_par == 1, -1.0, 1.0).astype(f32)
    dn = (((1,), (0,)), ((), ()))
    hp = lax.Precision.HIGHEST

    hiota = lax.broadcasted_iota(jnp.int32, (_NA // 8, 8, _BBLK), 0)
    pj = lax.population_count(hiota >> 3)
    podd_p = (pj & 1) == 1
    pb1 = (pj & 2) == 2
    sb1 = jnp.where(pb1, -1.0, 1.0).astype(f32)
    sxor = jnp.where(podd_p ^ pb1, -1.0, 1.0).astype(f32)

    def twiddle(rr, ii, sa, sb):
        r3 = rr.reshape(_NA // 8, 8, _BBLK)
        i3 = ii.reshape(_NA // 8, 8, _BBLK)
        tr = jnp.where(podd_p, i3, r3) * sa
        ti = jnp.where(podd_p, r3, i3) * sb
        return tr.reshape(_NA, _BBLK), ti.reshape(_NA, _BBLK)

    for l in range(_NL):
        for w in range(6, _NQ):
            r, im = rx(r, im, cq[l, w], sq[l, w], _NQ - 1 - w)
        amag = jnp.float32(1.0)
        for w in range(6):
            amag = amag * jnp.where(xbits[w], sq[l, w], cq[l, w])
        breal = amag * sgnb
        r, im = twiddle(r, im, sxor, sb1)
        r3 = r.reshape(64, 64, _BBLK)
        i3 = im.reshape(64, 64, _BBLK)
        nr = lax.dot_general(breal, r3, dn, precision=hp,
                             preferred_element_type=f32).reshape(_NA, _BBLK)
        ni = lax.dot_general(breal, i3, dn, precision=hp,
                             preferred_element_type=f32).reshape(_NA, _BBLK)
        r, im = twiddle(nr, ni, sb1, sxor)
        for w in range(_NQ):
            r = cnot_comp(r, w)
            im = cnot_comp(im, w)

    probs = r * r + im * im

    col = lax.broadcasted_iota(jnp.int32, (_NQ, _NA), 1)
    row = lax.broadcasted_iota(jnp.int32, (_NQ, _NA), 0)
    bitpos = jnp.right_shift(jnp.int32(_NA // 2), row)
    sgn = jnp.where((col & bitpos) != 0, -1.0, 1.0).astype(f32)
    z = jnp.dot(sgn, probs, preferred_element_type=f32)
    logits = jnp.dot(w_ref[...], z, preferred_element_type=f32)
    out_ref[...] = logits + b_ref[...]


def kernel(x, qweights, W, b):
    xt = x.T
    b2 = b.reshape(_NC, 1)
    out = pl.pallas_call(
        _qnn_body,
        grid=(_B // _BBLK,),
        in_specs=[
            pl.BlockSpec((_NQ, _BBLK), lambda i: (0, i)),
            pl.BlockSpec((_NL, _NQ), lambda i: (0, 0)),
            pl.BlockSpec((_NC, _NQ), lambda i: (0, 0)),
            pl.BlockSpec((_NC, 1), lambda i: (0, 0)),
        ],
        out_specs=pl.BlockSpec((_NC, _BBLK), lambda i: (0, i)),
        out_shape=jax.ShapeDtypeStruct((_NC, _B), jnp.float32),
        compiler_params=pltpu.CompilerParams(
            dimension_semantics=("parallel",),
            vmem_limit_bytes=100 * 1024 * 1024,
        ),
    )(xt, qweights, W, b2)
    return out.T


# speedup vs baseline: 1.2936x; 1.0203x over previous
import jax
import jax.numpy as jnp
from jax import lax
from jax.experimental import pallas as pl
from jax.experimental.pallas import tpu as pltpu

_NQ = 12
_NL = 6
_NC = 64
_B = 512
_BBLK = 256
_NA = 1 << _NQ


def _qnn_body(xt_ref, qw_ref, w_ref, b_ref, out_ref):
    f32 = jnp.float32

    xh = xt_ref[...] * 0.5
    cx = jnp.cos(xh)
    sx = jnp.sin(xh)
    qh = qw_ref[...] * 0.5
    cq = jnp.cos(qh)
    sq = jnp.sin(qh)

    miota = lax.broadcasted_iota(jnp.int32, (1, 8, _BBLK), 1)
    mb0 = (miota & 1) == 1
    mb1 = (miota & 2) == 2
    mb2 = (miota & 4) == 4
    mb = (mb0, mb1, mb2)

    def partner(a, bit):
        if bit >= 3:
            s = 1 << bit
            y = a.reshape(_NA // (2 * s), 2, s, _BBLK)
            p = jnp.concatenate([y[:, 1:2], y[:, 0:1]], axis=1)
            return p.reshape(_NA, _BBLK)
        y = a.reshape(_NA // 8, 8, _BBLK)
        if bit == 2:
            p = jnp.roll(y, 4, axis=1)
        else:
            s = 1 << bit
            p = jnp.where(mb[bit], jnp.roll(y, s, axis=1),
                          jnp.roll(y, -s, axis=1))
        return p.reshape(_NA, _BBLK)

    def rx(r, im, c, s, bit):
        pr = partner(r, bit)
        pi = partner(im, bit)
        return c * r + s * pi, c * im - s * pr

    def cnot_comp(a, w):
        if w <= 7:
            cb = 11 - w
            t = 1 << (cb - 1)
            y = a.reshape(_NA >> (cb + 1), 4, t, _BBLK)
            p = jnp.concatenate([y[:, 0:2], y[:, 3:4], y[:, 2:3]], axis=1)
            return p.reshape(_NA, _BBLK)
        if w == 8:
            y = a.reshape(_NA // 16, 2, 8, _BBLK)
            y1 = jnp.roll(y[:, 1], 4, axis=1).reshape(_NA // 16, 1, 8, _BBLK)
            return jnp.concatenate([y[:, 0:1], y1], axis=1).reshape(_NA, _BBLK)
        y = a.reshape(_NA // 8, 8, _BBLK)
        if w == 9:
            p = jnp.where(mb1, jnp.roll(y, 2, axis=1), jnp.roll(y, -2, axis=1))
            return jnp.where(mb2, p, y).reshape(_NA, _BBLK)
        if w == 10:
            p = jnp.where(mb0, jnp.roll(y, 1, axis=1), jnp.roll(y, -1, axis=1))
            return jnp.where(mb1, p, y).reshape(_NA, _BBLK)
        h = a.reshape(2, _NA // 16, 8, _BBLK)
        p = jnp.concatenate([h[1:2], h[0:1]], axis=0).reshape(_NA // 8, 8, _BBLK)
        return jnp.where(mb0, p, y).reshape(_NA, _BBLK)

    r = jnp.ones((1, _BBLK), f32)
    im = jnp.zeros((1, _BBLK), f32)
    for k in range(_NQ):
        w = _NQ - 1 - k
        c = cx[w:w + 1, :]
        s = sx[w:w + 1, :]
        nr = jnp.concatenate([c * r, s * im], axis=0)
        ni = jnp.concatenate([c * im, -(s * r)], axis=0)
        r, im = nr, ni

    jx = lax.broadcasted_iota(jnp.int32, (64, 64), 0)
    kx = lax.broadcasted_iota(jnp.int32, (64, 64), 1)
    xk = jx ^ kx
    xbits = [(xk >> (5 - w)) & 1 == 1 for w in range(6)]
    sgnb_par = (lax.population_count(kx) + lax.population_count(jx & kx)) & 1
    sgnb = jnp.where(sgnb_par == 1, -1.0, 1.0).astype(f32)
    dn = (((1,), (0,)), ((), ()))
    hp = lax.Precision.HIGHEST

    hiota = lax.broadcasted_iota(jnp.int32, (_NA // 8, 8, _BBLK), 0)
    pj = lax.population_count(hiota >> 3)
    podd_p = (pj & 1) == 1
    pb1 = (pj & 2) == 2
    sb1 = jnp.where(pb1, -1.0, 1.0).astype(f32)
    sxor = jnp.where(podd_p ^ pb1, -1.0, 1.0).astype(f32)

    def twiddle(rr, ii, sa, sb):
        r3 = rr.reshape(_NA // 8, 8, _BBLK)
        i3 = ii.reshape(_NA // 8, 8, _BBLK)
        tr = jnp.where(podd_p, i3, r3) * sa
        ti = jnp.where(podd_p, r3, i3) * sb
        return tr.reshape(_NA, _BBLK), ti.reshape(_NA, _BBLK)

    for l in range(_NL):
        for w in range(6, _NQ):
            r, im = rx(r, im, cq[l, w], sq[l, w], _NQ - 1 - w)
        amag = jnp.float32(1.0)
        for w in range(6):
            amag = amag * jnp.where(xbits[w], sq[l, w], cq[l, w])
        breal = amag * sgnb
        r, im = twiddle(r, im, sxor, sb1)
        r3 = r.reshape(64, 64, _BBLK)
        i3 = im.reshape(64, 64, _BBLK)
        nr = lax.dot_general(breal, r3, dn, precision=hp,
                             preferred_element_type=f32).reshape(_NA, _BBLK)
        ni = lax.dot_general(breal, i3, dn, precision=hp,
                             preferred_element_type=f32).reshape(_NA, _BBLK)
        if l < _NL - 1:
            r, im = twiddle(nr, ni, sb1, sxor)
            for w in range(_NQ):
                r = cnot_comp(r, w)
                im = cnot_comp(im, w)
        else:
            r, im = nr, ni

    probs = r * r + im * im

    col = lax.broadcasted_iota(jnp.int32, (_NQ, _NA), 1)
    row = lax.broadcasted_iota(jnp.int32, (_NQ, _NA), 0)
    suf = lax.population_count(jnp.right_shift(col, _NQ - 1 - row)) & 1
    p0 = (lax.population_count(col) + jnp.right_shift(col, _NQ - 1)) & 1
    parf = jnp.where(row == 0, p0, suf)
    sgn = jnp.where(parf == 1, -1.0, 1.0).astype(f32)
    z = jnp.dot(sgn, probs, preferred_element_type=f32)
    logits = jnp.dot(w_ref[...], z, preferred_element_type=f32)
    out_ref[...] = logits + b_ref[...]


def kernel(x, qweights, W, b):
    xt = x.T
    b2 = b.reshape(_NC, 1)
    out = pl.pallas_call(
        _qnn_body,
        grid=(_B // _BBLK,),
        in_specs=[
            pl.BlockSpec((_NQ, _BBLK), lambda i: (0, i)),
            pl.BlockSpec((_NL, _NQ), lambda i: (0, 0)),
            pl.BlockSpec((_NC, _NQ), lambda i: (0, 0)),
            pl.BlockSpec((_NC, 1), lambda i: (0, 0)),
        ],
        out_specs=pl.BlockSpec((_NC, _BBLK), lambda i: (0, i)),
        out_shape=jax.ShapeDtypeStruct((_NC, _B), jnp.float32),
        compiler_params=pltpu.CompilerParams(
            dimension_semantics=("parallel",),
            vmem_limit_bytes=100 * 1024 * 1024,
        ),
    )(xt, qweights, W, b2)
    return out.T
